# Initial kernel scaffold; baseline (speedup 1.0000x reference)
import jax
import jax.numpy as jnp
from jax.experimental import pallas as pl


def kernel(node_s, node_v, edge_index, edge_s, batch, Ws, bs, Wv, bv, We, be, Wm1, bm1, Wm2, bm2, Wo, bo, gamma, beta):
    raise NotImplementedError("write your pallas kernel here")


# speedup vs baseline: 1.5191x; 1.5191x over previous
import functools

import jax
import jax.numpy as jnp
from jax import lax
from jax.experimental import pallas as pl
from jax.experimental.pallas import tpu as pltpu
from jax.experimental.pallas import tpu_sc as plsc

N = 10000
E = 320000
B = 16
OUT = 256
H = 128
NT = 16
ET = E // NT
K = 16
NCH = ET // K
NPAD = 10240
NROWS_T = NPAD // NT
ZR = 128

f32 = jnp.float32


def _node_body(ns_ref, nvt_ref, ws_ref, bs_ref, wvb_ref, bv3_ref,
               w1a_ref, w1b_ref, w1c_ref, c0_ref,
               s_ref, vn_ref, a0_ref, a1_ref, b0_ref, b1_ref):
    s = jnp.dot(ns_ref[...], ws_ref[...], preferred_element_type=f32) + bs_ref[...]
    X = jnp.dot(nvt_ref[...], wvb_ref[...], preferred_element_type=f32) + bv3_ref[...]
    vn = jnp.sqrt(X[:, 0:16] ** 2 + X[:, 16:32] ** 2 + X[:, 32:48] ** 2)
    A = jnp.dot(s, w1a_ref[...], preferred_element_type=f32) + c0_ref[...]
    Bc = (jnp.dot(s, w1b_ref[...], preferred_element_type=f32)
          + jnp.dot(vn, w1c_ref[...], preferred_element_type=f32))
    s_ref[...] = s
    vn_ref[...] = vn
    a0_ref[...] = A[:, :H]
    a1_ref[...] = A[:, H:]
    b0_ref[...] = Bc[:, :H]
    b1_ref[...] = Bc[:, H:]


def _edge_c_body(es_ref, m_ref, c0o_ref, c1o_ref):
    Cf = jnp.dot(es_ref[...], m_ref[...], preferred_element_type=f32)
    c0o_ref[...] = Cf[:, :H]
    c1o_ref[...] = Cf[:, H:]


def _sc_edge(a0, a1, b0, b1, c0h, c1h, dsth, srch,
             s0o, s1o, cnto,
             idxd, idxs, abuf, bbuf, cbuf, onesb, zc16, ssh, csh, sem):
    c = lax.axis_index("c")
    sid = lax.axis_index("s")
    r0 = sid * NROWS_T

    zv = jnp.zeros((16,), f32)
    lane = lax.iota(jnp.int32, 16)
    onev = jnp.where(lane == 0, 1.0, 0.0).astype(f32)

    def zrow(i, _):
        for q in range(H // 16):
            abuf[i, pl.ds(q * 16, 16)] = zv
        return 0
    lax.fori_loop(0, K, zrow, 0)

    def zcrow(i, _):
        zc16[i, :] = zv
        return 0
    lax.fori_loop(0, ZR, zcrow, 0)

    def orow(i, _):
        onesb[i, :] = onev
        return 0
    lax.fori_loop(0, K, orow, 0)

    for k8 in range(NROWS_T // K):
        pltpu.sync_copy(abuf, ssh.at[pl.ds(r0 + k8 * K, K)])
    for k5 in range(NROWS_T // ZR):
        pltpu.sync_copy(zc16, csh.at[pl.ds(r0 + k5 * ZR, ZR)])
    plsc.subcore_barrier()

    def run_edges(ah, bh, ch, with_cnt):
        def body(j, _):
            base = sid * ET + j * K
            pltpu.sync_copy(dsth.at[pl.ds(base, K)], idxd)
            pltpu.sync_copy(srch.at[pl.ds(base, K)], idxs)
            cpa = pltpu.async_copy(ah.at[idxd], abuf, sem)
            cpb = pltpu.async_copy(bh.at[idxs], bbuf, sem)
            cpc = pltpu.async_copy(ch.at[pl.ds(base, K)], cbuf, sem)
            cpa.wait()
            cpb.wait()
            cpc.wait()

            def crow(r, _):
                for q in range(H // 16):
                    sl = pl.ds(q * 16, 16)
                    cbuf[r, sl] = jnp.maximum(abuf[r, sl] + bbuf[r, sl] + cbuf[r, sl], 0.0)
                return 0
            lax.fori_loop(0, K, crow, 0)

            pltpu.sync_copy(cbuf, ssh.at[idxd], add=True)
            if with_cnt:
                pltpu.sync_copy(onesb, csh.at[idxd], add=True)
            return 0
        lax.fori_loop(0, NCH, body, 0)

    @pl.when(c == 0)
    def _():
        run_edges(a0, b0, c0h, True)

    @pl.when(c == 1)
    def _():
        run_edges(a1, b1, c1h, False)

    plsc.subcore_barrier()

    @pl.when(c == 0)
    def _():
        for k5 in range(NROWS_T // ZR):
            sl = pl.ds(r0 + k5 * ZR, ZR)
            pltpu.sync_copy(ssh.at[sl], s0o.at[sl])
        pltpu.sync_copy(csh.at[pl.ds(r0, NROWS_T)], cnto.at[pl.ds(r0, NROWS_T)])

    @pl.when(c == 1)
    def _():
        for k5 in range(NROWS_T // ZR):
            sl = pl.ds(r0 + k5 * ZR, ZR)
            pltpu.sync_copy(ssh.at[sl], s1o.at[sl])


def _post_body(s0_ref, s1_ref, cnt_ref, s_ref, vn_ref, bt_ref,
               wm2t_ref, wm2b_ref, bm2_ref, wot_ref, wob_ref, bo_ref,
               g_ref, be_ref, out_ref, accp1, accp2, accg, *, nsteps, nb):
    i = pl.program_id(0)

    @pl.when(i == 0)
    def _():
        accp1[...] = jnp.zeros_like(accp1)
        accp2[...] = jnp.zeros_like(accp2)
        accg[...] = jnp.zeros_like(accg)

    SW = (jnp.dot(s0_ref[...], wm2t_ref[...], preferred_element_type=f32)
          + jnp.dot(s1_ref[...], wm2b_ref[...], preferred_element_type=f32))
    cnt = cnt_ref[...][:, 0:1]
    aggr = (SW + cnt * bm2_ref[...]) / jnp.maximum(cnt, 1.0)
    sn = s_ref[...] + aggr

    oh = (bt_ref[...] == lax.broadcasted_iota(jnp.int32, (nb, B), 1)).astype(f32)
    accp1[...] += lax.dot_general(oh, sn, (((0,), (0,)), ((), ())),
                                  preferred_element_type=f32)
    accp2[...] += lax.dot_general(oh, vn_ref[...], (((0,), (0,)), ((), ())),
                                  preferred_element_type=f32)
    accg[...] += lax.dot_general(oh, jnp.ones((nb, 8), f32), (((0,), (0,)), ((), ())),
                                 preferred_element_type=f32)

    @pl.when(i == nsteps - 1)
    def _():
        gc = jnp.maximum(accg[...][:, 0:1], 1.0)
        gv1 = accp1[...] / gc
        gv2 = accp2[...] / gc
        h = (jnp.dot(gv1, wot_ref[...], preferred_element_type=f32)
             + jnp.dot(gv2, wob_ref[...], preferred_element_type=f32)
             + bo_ref[...])
        mu = jnp.mean(h, axis=-1, keepdims=True)
        dd = h - mu
        var = jnp.mean(dd * dd, axis=-1, keepdims=True)
        hn = dd * lax.rsqrt(var + 1e-5) * g_ref[...] + be_ref[...]
        out_ref[...] = jnp.maximum(hn, 0.0)


def kernel(node_s, node_v, edge_index, edge_s, batch, Ws, bs, Wv, bv, We, be,
           Wm1, bm1, Wm2, bm2, Wo, bo, gamma, beta):
    W1a = Wm1[0:256]
    W1b = Wm1[256:512]
    W1c = Wm1[512:528]
    W1d = Wm1[528:784]
    M = We @ W1d
    c0 = (be @ W1d + bm1)[None, :]
    Wvblk = jax.scipy.linalg.block_diag(Wv, Wv, Wv)
    bv3 = jnp.tile(bv, 3)[None, :]
    nvt = jnp.transpose(node_v, (0, 2, 1)).reshape(N, 9)
    src = edge_index[0]
    dst = edge_index[1]
    bt2 = batch.reshape(N, 1)

    G1 = 5
    NB1 = N // G1
    full = lambda r, c_: pl.BlockSpec((r, c_), lambda i: (0, 0))
    rows = lambda r, c_: pl.BlockSpec((r, c_), lambda i: (i, 0))
    s_full, vn, A0, A1, B0, B1 = pl.pallas_call(
        _node_body,
        grid=(G1,),
        in_specs=[rows(NB1, 29), rows(NB1, 9), full(29, 256), full(1, 256),
                  full(9, 48), full(1, 48), full(256, 256), full(256, 256),
                  full(16, 256), full(1, 256)],
        out_specs=[rows(NB1, 256), rows(NB1, 16), rows(NB1, H), rows(NB1, H),
                   rows(NB1, H), rows(NB1, H)],
        out_shape=[jax.ShapeDtypeStruct((N, 256), f32),
                   jax.ShapeDtypeStruct((N, 16), f32),
                   jax.ShapeDtypeStruct((N, H), f32),
                   jax.ShapeDtypeStruct((N, H), f32),
                   jax.ShapeDtypeStruct((N, H), f32),
                   jax.ShapeDtypeStruct((N, H), f32)],
    )(node_s, nvt, Ws, bs[None, :], Wvblk, bv3, W1a, W1b, W1c, c0)

    G2 = 40
    EB = E // G2
    C0, C1 = pl.pallas_call(
        _edge_c_body,
        grid=(G2,),
        in_specs=[rows(EB, 5), full(5, 256)],
        out_specs=[rows(EB, H), rows(EB, H)],
        out_shape=[jax.ShapeDtypeStruct((E, H), f32),
                   jax.ShapeDtypeStruct((E, H), f32)],
    )(edge_s, M)

    sc_fn = pl.kernel(
        _sc_edge,
        out_type=[jax.ShapeDtypeStruct((NPAD, H), f32),
                  jax.ShapeDtypeStruct((NPAD, H), f32),
                  jax.ShapeDtypeStruct((NPAD, 16), f32)],
        mesh=plsc.VectorSubcoreMesh(core_axis_name="c", subcore_axis_name="s"),
        compiler_params=pltpu.CompilerParams(use_tc_tiling_on_sc=False),
        scratch_types=[pltpu.VMEM((K,), jnp.int32),
                       pltpu.VMEM((K,), jnp.int32),
                       pltpu.VMEM((K, H), f32),
                       pltpu.VMEM((K, H), f32),
                       pltpu.VMEM((K, H), f32),
                       pltpu.VMEM((K, 16), f32),
                       pltpu.VMEM((ZR, 16), f32),
                       pltpu.VMEM_SHARED((NPAD, H), f32),
                       pltpu.VMEM_SHARED((NPAD, 16), f32),
                       pltpu.SemaphoreType.DMA],
    )
    S0, S1, cntm = sc_fn(A0, A1, B0, B1, C0, C1, dst, src)
    S0, S1, cntm = S0[:N], S1[:N], cntm[:N]

    G4 = 5
    NB4 = N // G4
    out = pl.pallas_call(
        functools.partial(_post_body, nsteps=G4, nb=NB4),
        grid=(G4,),
        in_specs=[rows(NB4, H), rows(NB4, H), rows(NB4, 16), rows(NB4, 256),
                  rows(NB4, 16), rows(NB4, 1), full(H, 256), full(H, 256),
                  full(1, 256), full(256, 256), full(16, 256), full(1, 256),
                  full(1, 256), full(1, 256)],
        out_specs=pl.BlockSpec((B, 256), lambda i: (0, 0)),
        out_shape=jax.ShapeDtypeStruct((B, 256), f32),
        scratch_shapes=[pltpu.VMEM((B, 256), f32),
                        pltpu.VMEM((B, 16), f32),
                        pltpu.VMEM((B, 8), f32)],
    )(S0, S1, cntm, s_full, vn, bt2, Wm2[:H], Wm2[H:], bm2[None, :],
      Wo[:256], Wo[256:], bo[None, :], gamma[None, :], beta[None, :])
    return out


# speedup vs baseline: 2.9811x; 1.9625x over previous
import functools

import jax
import jax.numpy as jnp
from jax import lax
from jax.experimental import pallas as pl
from jax.experimental.pallas import tpu as pltpu
from jax.experimental.pallas import tpu_sc as plsc

N = 10000
E = 320000
B = 16
OUT = 256
Q = 64
NT = 16
ET = E // NT
K = 80
NCH = ET // K
NPAD = 10240
NROWS_T = NPAD // NT
ZR = 128

f32 = jnp.float32


def _node_body(ns_ref, nvt_ref, ws_ref, bs_ref, wvb_ref, bv3_ref,
               w1a_ref, w1b_ref, w1c_ref, c0_ref,
               s_ref, vn_ref, *aq_bq):
    s = jnp.dot(ns_ref[...], ws_ref[...], preferred_element_type=f32) + bs_ref[...]
    X = jnp.dot(nvt_ref[...], wvb_ref[...], preferred_element_type=f32) + bv3_ref[...]
    vn = jnp.sqrt(X[:, 0:16] ** 2 + X[:, 16:32] ** 2 + X[:, 32:48] ** 2)
    A = jnp.dot(s, w1a_ref[...], preferred_element_type=f32) + c0_ref[...]
    Bc = (jnp.dot(s, w1b_ref[...], preferred_element_type=f32)
          + jnp.dot(vn, w1c_ref[...], preferred_element_type=f32))
    s_ref[...] = s
    vn_ref[...] = vn
    for q in range(4):
        aq_bq[q][...] = A[:, q * Q:(q + 1) * Q]
        aq_bq[4 + q][...] = Bc[:, q * Q:(q + 1) * Q]


def _edge_c_body(es_ref, m_ref, *cq):
    Cf = jnp.dot(es_ref[...], m_ref[...], preferred_element_type=f32)
    for q in range(4):
        cq[q][...] = Cf[:, q * Q:(q + 1) * Q]


def _sc_edge(a00, a01, a10, a11, b00, b01, b10, b11, c00, c01, c10, c11,
             dst2, src2,
             s00, s01, s10, s11, cnto,
             idxd, idxs, ab0, bb0, cb0, ab1, bb1, cb1, onesb, zc16,
             ssh, csh, semA0, semB0, semC0, semA1, semB1, semC1):
    c = lax.axis_index("c")
    sid = lax.axis_index("s")
    r0 = sid * NROWS_T
    rowbase = sid * NCH

    zv = jnp.zeros((16,), f32)
    lane = lax.iota(jnp.int32, 16)
    onev = jnp.where(lane == 0, 1.0, 0.0).astype(f32)

    def zero_ab0():
        def zrow(i, _):
            for q in range(Q // 16):
                ab0[i, pl.ds(q * 16, 16)] = zv
            return 0
        lax.fori_loop(0, K, zrow, 0)

    def zero_own_rows():
        for k8 in range(NROWS_T // K):
            pltpu.sync_copy(ab0, ssh.at[pl.ds(r0 + k8 * K, K)])

    def zcrow(i, _):
        zc16[i, :] = zv
        return 0
    lax.fori_loop(0, ZR, zcrow, 0)

    def orow(i, _):
        onesb[i, :] = onev
        return 0
    lax.fori_loop(0, K, orow, 0)

    zero_ab0()
    zero_own_rows()
    for k5 in range(NROWS_T // ZR):
        pltpu.sync_copy(zc16, csh.at[pl.ds(r0 + k5 * ZR, ZR)])

    pltpu.sync_copy(dst2.at[pl.ds(rowbase, NCH)], idxd)
    pltpu.sync_copy(src2.at[pl.ds(rowbase, NCH)], idxs)
    plsc.subcore_barrier()

    slots = ((ab0, bb0, cb0, semA0, semB0, semC0),
             (ab1, bb1, cb1, semA1, semB1, semC1))

    def run_pass(ah, bh, ch, with_cnt):
        def issue(ck, sl):
            ab, bb, cb, sa, sb, sc_ = slots[sl]
            pltpu.async_copy(ah.at[idxd.at[ck]], ab, sa)
            pltpu.async_copy(bh.at[idxs.at[ck]], bb, sb)
            pltpu.async_copy(ch.at[pl.ds((rowbase + ck) * K, K)], cb, sc_)

        def proc(ck, sl):
            ab, bb, cb, sa, sb, sc_ = slots[sl]
            pltpu.make_async_copy(ah.at[idxd.at[0]], ab, sa).wait()
            pltpu.make_async_copy(bh.at[idxs.at[0]], bb, sb).wait()
            pltpu.make_async_copy(ch.at[pl.ds(0, K)], cb, sc_).wait()

            def crow(r, _):
                for q in range(Q // 16):
                    s2 = pl.ds(q * 16, 16)
                    cb[r, s2] = jnp.maximum(ab[r, s2] + bb[r, s2] + cb[r, s2], 0.0)
                return 0
            lax.fori_loop(0, K, crow, 0)

            pltpu.sync_copy(cb, ssh.at[idxd.at[ck]], add=True)
            if with_cnt:
                pltpu.sync_copy(onesb, csh.at[idxd.at[ck]], add=True)

        issue(0, 0)
        issue(1, 1)

        def body(jj, _):
            ck = 2 * jj
            proc(ck, 0)

            @pl.when(ck + 2 < NCH)
            def _():
                issue(ck + 2, 0)

            proc(ck + 1, 1)

            @pl.when(ck + 3 < NCH)
            def _():
                issue(ck + 3, 1)
            return 0
        lax.fori_loop(0, NCH // 2, body, 0)

    def writeout(so):
        for k8 in range(NROWS_T // K):
            slr = pl.ds(r0 + k8 * K, K)
            pltpu.sync_copy(ssh.at[slr], so.at[slr])

    @pl.when(c == 0)
    def _():
        run_pass(a00, b00, c00, True)

    @pl.when(c == 1)
    def _():
        run_pass(a10, b10, c10, False)

    plsc.subcore_barrier()

    @pl.when(c == 0)
    def _():
        writeout(s00)
        pltpu.sync_copy(csh.at[pl.ds(r0, NROWS_T)], cnto.at[pl.ds(r0, NROWS_T)])

    @pl.when(c == 1)
    def _():
        writeout(s10)

    zero_ab0()
    zero_own_rows()
    plsc.subcore_barrier()

    @pl.when(c == 0)
    def _():
        run_pass(a01, b01, c01, False)

    @pl.when(c == 1)
    def _():
        run_pass(a11, b11, c11, False)

    plsc.subcore_barrier()

    @pl.when(c == 0)
    def _():
        writeout(s01)

    @pl.when(c == 1)
    def _():
        writeout(s11)


def _post_body(s00_ref, s01_ref, s10_ref, s11_ref, cnt_ref, s_ref, vn_ref, bt_ref,
               w2a_ref, w2b_ref, w2c_ref, w2d_ref, bm2_ref, wot_ref, wob_ref,
               bo_ref, g_ref, be_ref, out_ref, accp1, accp2, accg, *, nsteps, nb):
    i = pl.program_id(0)

    @pl.when(i == 0)
    def _():
        accp1[...] = jnp.zeros_like(accp1)
        accp2[...] = jnp.zeros_like(accp2)
        accg[...] = jnp.zeros_like(accg)

    SW = (jnp.dot(s00_ref[...], w2a_ref[...], preferred_element_type=f32)
          + jnp.dot(s01_ref[...], w2b_ref[...], preferred_element_type=f32)
          + jnp.dot(s10_ref[...], w2c_ref[...], preferred_element_type=f32)
          + jnp.dot(s11_ref[...], w2d_ref[...], preferred_element_type=f32))
    cnt = cnt_ref[...][:, 0:1]
    aggr = (SW + cnt * bm2_ref[...]) / jnp.maximum(cnt, 1.0)
    sn = s_ref[...] + aggr

    oh = (bt_ref[...] == lax.broadcasted_iota(jnp.int32, (nb, B), 1)).astype(f32)
    accp1[...] += lax.dot_general(oh, sn, (((0,), (0,)), ((), ())),
                                  preferred_element_type=f32)
    accp2[...] += lax.dot_general(oh, vn_ref[...], (((0,), (0,)), ((), ())),
                                  preferred_element_type=f32)
    accg[...] += lax.dot_general(oh, jnp.ones((nb, 8), f32), (((0,), (0,)), ((), ())),
                                 preferred_element_type=f32)

    @pl.when(i == nsteps - 1)
    def _():
        gc = jnp.maximum(accg[...][:, 0:1], 1.0)
        gv1 = accp1[...] / gc
        gv2 = accp2[...] / gc
        h = (jnp.dot(gv1, wot_ref[...], preferred_element_type=f32)
             + jnp.dot(gv2, wob_ref[...], preferred_element_type=f32)
             + bo_ref[...])
        mu = jnp.mean(h, axis=-1, keepdims=True)
        dd = h - mu
        var = jnp.mean(dd * dd, axis=-1, keepdims=True)
        hn = dd * lax.rsqrt(var + 1e-5) * g_ref[...] + be_ref[...]
        out_ref[...] = jnp.maximum(hn, 0.0)


def kernel(node_s, node_v, edge_index, edge_s, batch, Ws, bs, Wv, bv, We, be,
           Wm1, bm1, Wm2, bm2, Wo, bo, gamma, beta):
    W1a = Wm1[0:256]
    W1b = Wm1[256:512]
    W1c = Wm1[512:528]
    W1d = Wm1[528:784]
    M = We @ W1d
    c0 = (be @ W1d + bm1)[None, :]
    Wvblk = jax.scipy.linalg.block_diag(Wv, Wv, Wv)
    bv3 = jnp.tile(bv, 3)[None, :]
    nvt = jnp.transpose(node_v, (0, 2, 1)).reshape(N, 9)
    src2 = edge_index[0].reshape(E // K, K)
    dst2 = edge_index[1].reshape(E // K, K)
    bt2 = batch.reshape(N, 1)

    G1 = 5
    NB1 = N // G1
    full = lambda r, c_: pl.BlockSpec((r, c_), lambda i: (0, 0))
    rows = lambda r, c_: pl.BlockSpec((r, c_), lambda i: (i, 0))
    nq = jax.ShapeDtypeStruct((N, Q), f32)
    s_full, vn, A00, A01, A10, A11, B00, B01, B10, B11 = pl.pallas_call(
        _node_body,
        grid=(G1,),
        in_specs=[rows(NB1, 29), rows(NB1, 9), full(29, 256), full(1, 256),
                  full(9, 48), full(1, 48), full(256, 256), full(256, 256),
                  full(16, 256), full(1, 256)],
        out_specs=[rows(NB1, 256), rows(NB1, 16)] + [rows(NB1, Q)] * 8,
        out_shape=[jax.ShapeDtypeStruct((N, 256), f32),
                   jax.ShapeDtypeStruct((N, 16), f32)] + [nq] * 8,
    )(node_s, nvt, Ws, bs[None, :], Wvblk, bv3, W1a, W1b, W1c, c0)

    G2 = 40
    EB = E // G2
    eq = jax.ShapeDtypeStruct((E, Q), f32)
    C00, C01, C10, C11 = pl.pallas_call(
        _edge_c_body,
        grid=(G2,),
        in_specs=[rows(EB, 5), full(5, 256)],
        out_specs=[rows(EB, Q)] * 4,
        out_shape=[eq] * 4,
    )(edge_s, M)

    sq = jax.ShapeDtypeStruct((NPAD, Q), f32)
    sc_fn = pl.kernel(
        _sc_edge,
        out_type=[sq, sq, sq, sq, jax.ShapeDtypeStruct((NPAD, 16), f32)],
        mesh=plsc.VectorSubcoreMesh(core_axis_name="c", subcore_axis_name="s"),
        compiler_params=pltpu.CompilerParams(use_tc_tiling_on_sc=False),
        scratch_types=[pltpu.VMEM((NCH, K), jnp.int32),
                       pltpu.VMEM((NCH, K), jnp.int32),
                       pltpu.VMEM((K, Q), f32),
                       pltpu.VMEM((K, Q), f32),
                       pltpu.VMEM((K, Q), f32),
                       pltpu.VMEM((K, Q), f32),
                       pltpu.VMEM((K, Q), f32),
                       pltpu.VMEM((K, Q), f32),
                       pltpu.VMEM((K, 16), f32),
                       pltpu.VMEM((ZR, 16), f32),
                       pltpu.VMEM_SHARED((NPAD, Q), f32),
                       pltpu.VMEM_SHARED((NPAD, 16), f32),
                       pltpu.SemaphoreType.DMA,
                       pltpu.SemaphoreType.DMA,
                       pltpu.SemaphoreType.DMA,
                       pltpu.SemaphoreType.DMA,
                       pltpu.SemaphoreType.DMA,
                       pltpu.SemaphoreType.DMA],
    )
    S00, S01, S10, S11, cntm = sc_fn(A00, A01, A10, A11, B00, B01, B10, B11,
                                     C00, C01, C10, C11, dst2, src2)
    S00, S01, S10, S11 = S00[:N], S01[:N], S10[:N], S11[:N]
    cntm = cntm[:N]

    G4 = 5
    NB4 = N // G4
    out = pl.pallas_call(
        functools.partial(_post_body, nsteps=G4, nb=NB4),
        grid=(G4,),
        in_specs=[rows(NB4, Q)] * 4 + [rows(NB4, 16), rows(NB4, 256),
                  rows(NB4, 16), rows(NB4, 1), full(Q, 256), full(Q, 256),
                  full(Q, 256), full(Q, 256), full(1, 256), full(256, 256),
                  full(16, 256), full(1, 256), full(1, 256), full(1, 256)],
        out_specs=pl.BlockSpec((B, 256), lambda i: (0, 0)),
        out_shape=jax.ShapeDtypeStruct((B, 256), f32),
        scratch_shapes=[pltpu.VMEM((B, 256), f32),
                        pltpu.VMEM((B, 16), f32),
                        pltpu.VMEM((B, 8), f32)],
    )(S00, S01, S10, S11, cntm, s_full, vn, bt2,
      Wm2[0:Q], Wm2[Q:2 * Q], Wm2[2 * Q:3 * Q], Wm2[3 * Q:], bm2[None, :],
      Wo[:256], Wo[256:], bo[None, :], gamma[None, :], beta[None, :])
    return out


# speedup vs baseline: 4.3648x; 1.4641x over previous
import functools

import jax
import jax.numpy as jnp
from jax import lax
from jax.experimental import pallas as pl
from jax.experimental.pallas import tpu as pltpu
from jax.experimental.pallas import tpu_sc as plsc

N = 10000
E = 320000
B = 16
OUT = 256
H = 128
NT = 16
ET = E // NT
KB = 40
NSC = ET // KB
NPAD = 10240
NROWS_T = NPAD // NT

f32 = jnp.float32


def _node_body(ns_ref, nvt_ref, ws_ref, bs_ref, wvb_ref, bv3_ref,
               w1a_ref, w1b_ref, w1c_ref, c0_ref,
               s_ref, vn_ref, a0_ref, a1_ref, b0_ref, b1_ref):
    s = jnp.dot(ns_ref[...], ws_ref[...], preferred_element_type=f32) + bs_ref[...]
    X = jnp.dot(nvt_ref[...], wvb_ref[...], preferred_element_type=f32) + bv3_ref[...]
    vn = jnp.sqrt(X[:, 0:16] ** 2 + X[:, 16:32] ** 2 + X[:, 32:48] ** 2)
    A = jnp.dot(s, w1a_ref[...], preferred_element_type=f32) + c0_ref[...]
    Bc = (jnp.dot(s, w1b_ref[...], preferred_element_type=f32)
          + jnp.dot(vn, w1c_ref[...], preferred_element_type=f32))
    s_ref[...] = s
    vn_ref[...] = vn
    a0_ref[...] = A[:, :H]
    a1_ref[...] = A[:, H:]
    b0_ref[...] = Bc[:, :H]
    b1_ref[...] = Bc[:, H:]


def _edge_c_body(es_ref, m_ref, c0o_ref, c1o_ref):
    Cf = jnp.dot(es_ref[...], m_ref[...], preferred_element_type=f32)
    c0o_ref[...] = Cf[:, :H]
    c1o_ref[...] = Cf[:, H:]


def _sc_edge(a0, a1, b0, b1, c0h, c1h, dsth, srch, zh, zc8h, ones2h,
             s0o, s1o, cnto,
             idxd0, idxs0, idxd1, idxs1, ab0, bb0, cb0, ab1, bb1, cb1, onesv,
             ssh, csh, semA0, semB0, semC0, semA1, semB1, semC1, semI0, semI1):
    c = lax.axis_index("c")
    sid = lax.axis_index("s")
    r0 = sid * NROWS_T
    ebase = sid * ET

    pltpu.sync_copy(zh, ssh.at[pl.ds(r0, NROWS_T)])
    pltpu.sync_copy(zc8h, csh.at[pl.ds(r0, NROWS_T)])
    pltpu.sync_copy(ones2h, onesv)
    plsc.subcore_barrier()

    islots = ((idxd0, idxs0, semI0), (idxd1, idxs1, semI1))
    bslots = ((ab0, bb0, cb0, semA0, semB0, semC0),
              (ab1, bb1, cb1, semA1, semB1, semC1))

    def issue_idx(k, sl):
        idd, ids, si = islots[sl]
        esl = pl.ds(ebase + k * KB, KB)
        pltpu.async_copy(dsth.at[esl], idd, si)
        pltpu.async_copy(srch.at[esl], ids, si)

    def wait_idx(sl):
        idd, ids, si = islots[sl]
        pltpu.make_async_copy(dsth.at[pl.ds(0, KB)], idd, si).wait()
        pltpu.make_async_copy(srch.at[pl.ds(0, KB)], ids, si).wait()

    def run_half(ah, bh, ch, with_cnt):
        def issue(k, sl):
            ab, bb, cb, sa, sb, sc_ = bslots[sl]
            idd, ids, _ = islots[sl]
            pltpu.async_copy(ah.at[idd], ab, sa)
            pltpu.async_copy(bh.at[ids], bb, sb)
            pltpu.async_copy(ch.at[pl.ds(ebase + k * KB, KB)], cb, sc_)

        def proc(k, sl):
            ab, bb, cb, sa, sb, sc_ = bslots[sl]
            idd, ids, _ = islots[sl]
            pltpu.make_async_copy(ah.at[idd], ab, sa).wait()
            pltpu.make_async_copy(bh.at[ids], bb, sb).wait()
            pltpu.make_async_copy(ch.at[pl.ds(0, KB)], cb, sc_).wait()

            def crow(r, _):
                for q in range(H // 16):
                    s2 = pl.ds(q * 16, 16)
                    cb[r, s2] = jnp.maximum(
                        ab[r, s2] + bb[r, s2] + cb[r, s2], 0.0)
                return 0
            lax.fori_loop(0, KB, crow, 0)

            pltpu.sync_copy(cb, ssh.at[idd], add=True)
            if with_cnt:
                pltpu.sync_copy(onesv, csh.at[idd], add=True)

        issue_idx(0, 0)
        issue_idx(1, 1)
        wait_idx(0)
        issue(0, 0)
        wait_idx(1)
        issue(1, 1)

        def body(jj, _):
            k0 = 2 * jj
            proc(k0, 0)

            @pl.when(k0 + 2 < NSC)
            def _():
                issue_idx(k0 + 2, 0)
                wait_idx(0)
                issue(k0 + 2, 0)

            proc(k0 + 1, 1)

            @pl.when(k0 + 3 < NSC)
            def _():
                issue_idx(k0 + 3, 1)
                wait_idx(1)
                issue(k0 + 3, 1)
            return 0
        lax.fori_loop(0, NSC // 2, body, 0)

    @pl.when(c == 0)
    def _():
        run_half(a0, b0, c0h, True)

    @pl.when(c == 1)
    def _():
        run_half(a1, b1, c1h, False)

    plsc.subcore_barrier()

    @pl.when(c == 0)
    def _():
        pltpu.sync_copy(ssh.at[pl.ds(r0, NROWS_T)], s0o.at[pl.ds(r0, NROWS_T)])
        pltpu.sync_copy(csh.at[pl.ds(r0, NROWS_T)], cnto.at[pl.ds(r0, NROWS_T)])

    @pl.when(c == 1)
    def _():
        pltpu.sync_copy(ssh.at[pl.ds(r0, NROWS_T)], s1o.at[pl.ds(r0, NROWS_T)])


def _post_body(s0_ref, s1_ref, cnt_ref, s_ref, vn_ref, bt_ref,
               wm2t_ref, wm2b_ref, bm2_ref, wot_ref, wob_ref, bo_ref,
               g_ref, be_ref, out_ref, accp1, accp2, accg, *, nsteps, nb):
    i = pl.program_id(0)

    @pl.when(i == 0)
    def _():
        accp1[...] = jnp.zeros_like(accp1)
        accp2[...] = jnp.zeros_like(accp2)
        accg[...] = jnp.zeros_like(accg)

    SW = (jnp.dot(s0_ref[...], wm2t_ref[...], preferred_element_type=f32)
          + jnp.dot(s1_ref[...], wm2b_ref[...], preferred_element_type=f32))
    cnt = cnt_ref[...][:, 0:1]
    aggr = (SW + cnt * bm2_ref[...]) / jnp.maximum(cnt, 1.0)
    sn = s_ref[...] + aggr

    oh = (bt_ref[...] == lax.broadcasted_iota(jnp.int32, (nb, B), 1)).astype(f32)
    accp1[...] += lax.dot_general(oh, sn, (((0,), (0,)), ((), ())),
                                  preferred_element_type=f32)
    accp2[...] += lax.dot_general(oh, vn_ref[...], (((0,), (0,)), ((), ())),
                                  preferred_element_type=f32)
    accg[...] += lax.dot_general(oh, jnp.ones((nb, 8), f32), (((0,), (0,)), ((), ())),
                                 preferred_element_type=f32)

    @pl.when(i == nsteps - 1)
    def _():
        gc = jnp.maximum(accg[...][:, 0:1], 1.0)
        gv1 = accp1[...] / gc
        gv2 = accp2[...] / gc
        h = (jnp.dot(gv1, wot_ref[...], preferred_element_type=f32)
             + jnp.dot(gv2, wob_ref[...], preferred_element_type=f32)
             + bo_ref[...])
        mu = jnp.mean(h, axis=-1, keepdims=True)
        dd = h - mu
        var = jnp.mean(dd * dd, axis=-1, keepdims=True)
        hn = dd * lax.rsqrt(var + 1e-5) * g_ref[...] + be_ref[...]
        out_ref[...] = jnp.maximum(hn, 0.0)


def kernel(node_s, node_v, edge_index, edge_s, batch, Ws, bs, Wv, bv, We, be,
           Wm1, bm1, Wm2, bm2, Wo, bo, gamma, beta):
    W1a = Wm1[0:256]
    W1b = Wm1[256:512]
    W1c = Wm1[512:528]
    W1d = Wm1[528:784]
    M = We @ W1d
    c0 = (be @ W1d + bm1)[None, :]
    Wvblk = jax.scipy.linalg.block_diag(Wv, Wv, Wv)
    bv3 = jnp.tile(bv, 3)[None, :]
    nvt = jnp.transpose(node_v, (0, 2, 1)).reshape(N, 9)
    src = edge_index[0]
    dst = edge_index[1]
    bt2 = batch.reshape(N, 1)

    G1 = 5
    NB1 = N // G1
    full = lambda r, c_: pl.BlockSpec((r, c_), lambda i: (0, 0))
    rows = lambda r, c_: pl.BlockSpec((r, c_), lambda i: (i, 0))
    s_full, vn, A0, A1, B0, B1 = pl.pallas_call(
        _node_body,
        grid=(G1,),
        in_specs=[rows(NB1, 29), rows(NB1, 9), full(29, 256), full(1, 256),
                  full(9, 48), full(1, 48), full(256, 256), full(256, 256),
                  full(16, 256), full(1, 256)],
        out_specs=[rows(NB1, 256), rows(NB1, 16), rows(NB1, H), rows(NB1, H),
                   rows(NB1, H), rows(NB1, H)],
        out_shape=[jax.ShapeDtypeStruct((N, 256), f32),
                   jax.ShapeDtypeStruct((N, 16), f32),
                   jax.ShapeDtypeStruct((N, H), f32),
                   jax.ShapeDtypeStruct((N, H), f32),
                   jax.ShapeDtypeStruct((N, H), f32),
                   jax.ShapeDtypeStruct((N, H), f32)],
    )(node_s, nvt, Ws, bs[None, :], Wvblk, bv3, W1a, W1b, W1c, c0)

    G2 = 40
    EB = E // G2
    C0, C1 = pl.pallas_call(
        _edge_c_body,
        grid=(G2,),
        in_specs=[rows(EB, 5), full(5, 256)],
        out_specs=[rows(EB, H), rows(EB, H)],
        out_shape=[jax.ShapeDtypeStruct((E, H), f32),
                   jax.ShapeDtypeStruct((E, H), f32)],
    )(edge_s, M)

    zzh = jnp.zeros((NROWS_T, H), f32)
    zc8 = jnp.zeros((NROWS_T, 8), f32)
    ones2 = jnp.zeros((KB, 8), f32).at[:, 0].set(1.0)
    sc_fn = pl.kernel(
        _sc_edge,
        out_type=[jax.ShapeDtypeStruct((NPAD, H), f32),
                  jax.ShapeDtypeStruct((NPAD, H), f32),
                  jax.ShapeDtypeStruct((NPAD, 8), f32)],
        mesh=plsc.VectorSubcoreMesh(core_axis_name="c", subcore_axis_name="s"),
        compiler_params=pltpu.CompilerParams(use_tc_tiling_on_sc=False),
        scratch_types=[pltpu.VMEM((KB,), jnp.int32),
                       pltpu.VMEM((KB,), jnp.int32),
                       pltpu.VMEM((KB,), jnp.int32),
                       pltpu.VMEM((KB,), jnp.int32),
                       pltpu.VMEM((KB, H), f32),
                       pltpu.VMEM((KB, H), f32),
                       pltpu.VMEM((KB, H), f32),
                       pltpu.VMEM((KB, H), f32),
                       pltpu.VMEM((KB, H), f32),
                       pltpu.VMEM((KB, H), f32),
                       pltpu.VMEM((KB, 8), f32),
                       pltpu.VMEM_SHARED((NPAD, H), f32),
                       pltpu.VMEM_SHARED((NPAD, 8), f32),
                       pltpu.SemaphoreType.DMA,
                       pltpu.SemaphoreType.DMA,
                       pltpu.SemaphoreType.DMA,
                       pltpu.SemaphoreType.DMA,
                       pltpu.SemaphoreType.DMA,
                       pltpu.SemaphoreType.DMA,
                       pltpu.SemaphoreType.DMA,
                       pltpu.SemaphoreType.DMA],
    )
    S0, S1, cntm = sc_fn(A0, A1, B0, B1, C0, C1, dst, src, zzh, zc8, ones2)
    S0, S1, cntm = S0[:N], S1[:N], cntm[:N]

    G4 = 5
    NB4 = N // G4
    out = pl.pallas_call(
        functools.partial(_post_body, nsteps=G4, nb=NB4),
        grid=(G4,),
        in_specs=[rows(NB4, H), rows(NB4, H), rows(NB4, 8), rows(NB4, 256),
                  rows(NB4, 16), rows(NB4, 1), full(H, 256), full(H, 256),
                  full(1, 256), full(256, 256), full(16, 256), full(1, 256),
                  full(1, 256), full(1, 256)],
        out_specs=pl.BlockSpec((B, 256), lambda i: (0, 0)),
        out_shape=jax.ShapeDtypeStruct((B, 256), f32),
        scratch_shapes=[pltpu.VMEM((B, 256), f32),
                        pltpu.VMEM((B, 16), f32),
                        pltpu.VMEM((B, 8), f32)],
    )(S0, S1, cntm, s_full, vn, bt2, Wm2[:H], Wm2[H:], bm2[None, :],
      Wo[:256], Wo[256:], bo[None, :], gamma[None, :], beta[None, :])
    return out


# speedup vs baseline: 5.0426x; 1.1553x over previous
import functools

import jax
import jax.numpy as jnp
from jax import lax
from jax.experimental import pallas as pl
from jax.experimental.pallas import tpu as pltpu
from jax.experimental.pallas import tpu_sc as plsc

N = 10000
E = 320000
B = 16
OUT = 256
H = 128
NT = 16
ET = E // NT
KB = 40
NSC = ET // KB
NPAD = 10240
NROWS_T = NPAD // NT

f32 = jnp.float32


def _node_body(ns_ref, nvt_ref, ws_ref, bs_ref, wvb_ref, bv3_ref,
               w1a_ref, w1b_ref, w1c_ref, c0_ref,
               s_ref, vn_ref, a0_ref, a1_ref, b0_ref, b1_ref):
    s = jnp.dot(ns_ref[...], ws_ref[...], preferred_element_type=f32) + bs_ref[...]
    X = jnp.dot(nvt_ref[...], wvb_ref[...], preferred_element_type=f32) + bv3_ref[...]
    vn = jnp.sqrt(X[:, 0:16] ** 2 + X[:, 16:32] ** 2 + X[:, 32:48] ** 2)
    A = jnp.dot(s, w1a_ref[...], preferred_element_type=f32) + c0_ref[...]
    Bc = (jnp.dot(s, w1b_ref[...], preferred_element_type=f32)
          + jnp.dot(vn, w1c_ref[...], preferred_element_type=f32))
    s_ref[...] = s
    vn_ref[...] = vn
    a0_ref[...] = A[:, :H]
    a1_ref[...] = A[:, H:]
    b0_ref[...] = Bc[:, :H]
    b1_ref[...] = Bc[:, H:]


def _edge_c_body(es_ref, m_ref, c0o_ref, c1o_ref):
    Cf = jnp.dot(es_ref[...], m_ref[...], preferred_element_type=f32)
    c0o_ref[...] = Cf[:, :H]
    c1o_ref[...] = Cf[:, H:]


def _sc_edge(a0, a1, b0, b1, c0h, c1h, dsth, srch, zh, zc8h, ones2h,
             s0o, s1o, cnto,
             idxd0, idxs0, idxd1, idxs1, idxd2, idxs2, idxd3, idxs3,
             ab0, bb0, cb0, ab1, bb1, cb1, onesv,
             ssh, csh, semA0, semB0, semC0, semA1, semB1, semC1,
             semI0, semI1, semI2, semI3):
    c = lax.axis_index("c")
    sid = lax.axis_index("s")
    r0 = sid * NROWS_T
    ebase = sid * ET

    pltpu.sync_copy(zh, ssh.at[pl.ds(r0, NROWS_T)])
    pltpu.sync_copy(zc8h, csh.at[pl.ds(r0, NROWS_T)])
    pltpu.sync_copy(ones2h, onesv)
    plsc.subcore_barrier()

    islots = ((idxd0, idxs0, semI0), (idxd1, idxs1, semI1),
              (idxd2, idxs2, semI2), (idxd3, idxs3, semI3))
    bslots = ((ab0, bb0, cb0, semA0, semB0, semC0),
              (ab1, bb1, cb1, semA1, semB1, semC1))

    def issue_idx(k, sl):
        idd, ids, si = islots[sl]
        esl = pl.ds(ebase + k * KB, KB)
        pltpu.async_copy(dsth.at[esl], idd, si)
        pltpu.async_copy(srch.at[esl], ids, si)

    def wait_idx(sl):
        idd, ids, si = islots[sl]
        pltpu.make_async_copy(dsth.at[pl.ds(0, KB)], idd, si).wait()
        pltpu.make_async_copy(srch.at[pl.ds(0, KB)], ids, si).wait()

    def run_half(ah, bh, ch, with_cnt):
        def issue(k, bsl, isl):
            ab, bb, cb, sa, sb, sc_ = bslots[bsl]
            idd, ids, _ = islots[isl]
            pltpu.async_copy(ah.at[idd], ab, sa)
            pltpu.async_copy(bh.at[ids], bb, sb)
            pltpu.async_copy(ch.at[pl.ds(ebase + k * KB, KB)], cb, sc_)

        def proc(k, bsl, isl):
            ab, bb, cb, sa, sb, sc_ = bslots[bsl]
            idd, ids, _ = islots[isl]
            pltpu.make_async_copy(ah.at[idd], ab, sa).wait()
            pltpu.make_async_copy(bh.at[ids], bb, sb).wait()
            pltpu.make_async_copy(ch.at[pl.ds(0, KB)], cb, sc_).wait()

            def crow(r, _):
                for q in range(H // 16):
                    s2 = pl.ds(q * 16, 16)
                    cb[r, s2] = jnp.maximum(
                        ab[r, s2] + bb[r, s2] + cb[r, s2], 0.0)
                return 0
            lax.fori_loop(0, KB, crow, 0)

            pltpu.sync_copy(cb, ssh.at[idd], add=True)
            if with_cnt:
                pltpu.sync_copy(onesv, csh.at[idd], add=True)

        for u in range(4):
            issue_idx(u, u)
        wait_idx(0)
        issue(0, 0, 0)
        wait_idx(1)
        issue(1, 1, 1)

        def body(jj, _):
            for u in range(4):
                ck = 4 * jj + u
                bsl = u % 2
                proc(ck, bsl, u)

                @pl.when(ck + 4 < NSC)
                def _():
                    issue_idx(ck + 4, u)

                @pl.when(ck + 2 < NSC)
                def _():
                    wait_idx((u + 2) % 4)
                    issue(ck + 2, bsl, (u + 2) % 4)
            return 0
        lax.fori_loop(0, NSC // 4, body, 0)

    @pl.when(c == 0)
    def _():
        run_half(a0, b0, c0h, True)

    @pl.when(c == 1)
    def _():
        run_half(a1, b1, c1h, False)

    plsc.subcore_barrier()

    @pl.when(c == 0)
    def _():
        pltpu.sync_copy(ssh.at[pl.ds(r0, NROWS_T)], s0o.at[pl.ds(r0, NROWS_T)])
        pltpu.sync_copy(csh.at[pl.ds(r0, NROWS_T)], cnto.at[pl.ds(r0, NROWS_T)])

    @pl.when(c == 1)
    def _():
        pltpu.sync_copy(ssh.at[pl.ds(r0, NROWS_T)], s1o.at[pl.ds(r0, NROWS_T)])


def _post_body(s0_ref, s1_ref, cnt_ref, s_ref, vn_ref, bt_ref,
               wm2t_ref, wm2b_ref, bm2_ref, wot_ref, wob_ref, bo_ref,
               g_ref, be_ref, out_ref, accp1, accp2, accg, *, nsteps, nb):
    i = pl.program_id(0)

    @pl.when(i == 0)
    def _():
        accp1[...] = jnp.zeros_like(accp1)
        accp2[...] = jnp.zeros_like(accp2)
        accg[...] = jnp.zeros_like(accg)

    SW = (jnp.dot(s0_ref[...], wm2t_ref[...], preferred_element_type=f32)
          + jnp.dot(s1_ref[...], wm2b_ref[...], preferred_element_type=f32))
    cnt = cnt_ref[...][:, 0:1]
    aggr = (SW + cnt * bm2_ref[...]) / jnp.maximum(cnt, 1.0)
    sn = s_ref[...] + aggr

    oh = (bt_ref[...] == lax.broadcasted_iota(jnp.int32, (nb, B), 1)).astype(f32)
    accp1[...] += lax.dot_general(oh, sn, (((0,), (0,)), ((), ())),
                                  preferred_element_type=f32)
    accp2[...] += lax.dot_general(oh, vn_ref[...], (((0,), (0,)), ((), ())),
                                  preferred_element_type=f32)
    accg[...] += lax.dot_general(oh, jnp.ones((nb, 8), f32), (((0,), (0,)), ((), ())),
                                 preferred_element_type=f32)

    @pl.when(i == nsteps - 1)
    def _():
        gc = jnp.maximum(accg[...][:, 0:1], 1.0)
        gv1 = accp1[...] / gc
        gv2 = accp2[...] / gc
        h = (jnp.dot(gv1, wot_ref[...], preferred_element_type=f32)
             + jnp.dot(gv2, wob_ref[...], preferred_element_type=f32)
             + bo_ref[...])
        mu = jnp.mean(h, axis=-1, keepdims=True)
        dd = h - mu
        var = jnp.mean(dd * dd, axis=-1, keepdims=True)
        hn = dd * lax.rsqrt(var + 1e-5) * g_ref[...] + be_ref[...]
        out_ref[...] = jnp.maximum(hn, 0.0)


def kernel(node_s, node_v, edge_index, edge_s, batch, Ws, bs, Wv, bv, We, be,
           Wm1, bm1, Wm2, bm2, Wo, bo, gamma, beta):
    W1a = Wm1[0:256]
    W1b = Wm1[256:512]
    W1c = Wm1[512:528]
    W1d = Wm1[528:784]
    M = We @ W1d
    c0 = (be @ W1d + bm1)[None, :]
    Wvblk = jax.scipy.linalg.block_diag(Wv, Wv, Wv)
    bv3 = jnp.tile(bv, 3)[None, :]
    nvt = jnp.transpose(node_v, (0, 2, 1)).reshape(N, 9)
    src = edge_index[0]
    dst = edge_index[1]
    bt2 = batch.reshape(N, 1)

    G1 = 5
    NB1 = N // G1
    full = lambda r, c_: pl.BlockSpec((r, c_), lambda i: (0, 0))
    rows = lambda r, c_: pl.BlockSpec((r, c_), lambda i: (i, 0))
    s_full, vn, A0, A1, B0, B1 = pl.pallas_call(
        _node_body,
        grid=(G1,),
        in_specs=[rows(NB1, 29), rows(NB1, 9), full(29, 256), full(1, 256),
                  full(9, 48), full(1, 48), full(256, 256), full(256, 256),
                  full(16, 256), full(1, 256)],
        out_specs=[rows(NB1, 256), rows(NB1, 16), rows(NB1, H), rows(NB1, H),
                   rows(NB1, H), rows(NB1, H)],
        out_shape=[jax.ShapeDtypeStruct((N, 256), f32),
                   jax.ShapeDtypeStruct((N, 16), f32),
                   jax.ShapeDtypeStruct((N, H), f32),
                   jax.ShapeDtypeStruct((N, H), f32),
                   jax.ShapeDtypeStruct((N, H), f32),
                   jax.ShapeDtypeStruct((N, H), f32)],
    )(node_s, nvt, Ws, bs[None, :], Wvblk, bv3, W1a, W1b, W1c, c0)

    G2 = 40
    EB = E // G2
    C0, C1 = pl.pallas_call(
        _edge_c_body,
        grid=(G2,),
        in_specs=[rows(EB, 5), full(5, 256)],
        out_specs=[rows(EB, H), rows(EB, H)],
        out_shape=[jax.ShapeDtypeStruct((E, H), f32),
                   jax.ShapeDtypeStruct((E, H), f32)],
    )(edge_s, M)

    zzh = jnp.zeros((NROWS_T, H), f32)
    zc8 = jnp.zeros((NROWS_T, 8), f32)
    ones2 = jnp.zeros((KB, 8), f32).at[:, 0].set(1.0)
    sc_fn = pl.kernel(
        _sc_edge,
        out_type=[jax.ShapeDtypeStruct((NPAD, H), f32),
                  jax.ShapeDtypeStruct((NPAD, H), f32),
                  jax.ShapeDtypeStruct((NPAD, 8), f32)],
        mesh=plsc.VectorSubcoreMesh(core_axis_name="c", subcore_axis_name="s"),
        compiler_params=pltpu.CompilerParams(use_tc_tiling_on_sc=False),
        scratch_types=[pltpu.VMEM((KB,), jnp.int32),
                       pltpu.VMEM((KB,), jnp.int32),
                       pltpu.VMEM((KB,), jnp.int32),
                       pltpu.VMEM((KB,), jnp.int32),
                       pltpu.VMEM((KB,), jnp.int32),
                       pltpu.VMEM((KB,), jnp.int32),
                       pltpu.VMEM((KB,), jnp.int32),
                       pltpu.VMEM((KB,), jnp.int32),
                       pltpu.VMEM((KB, H), f32),
                       pltpu.VMEM((KB, H), f32),
                       pltpu.VMEM((KB, H), f32),
                       pltpu.VMEM((KB, H), f32),
                       pltpu.VMEM((KB, H), f32),
                       pltpu.VMEM((KB, H), f32),
                       pltpu.VMEM((KB, 8), f32),
                       pltpu.VMEM_SHARED((NPAD, H), f32),
                       pltpu.VMEM_SHARED((NPAD, 8), f32),
                       pltpu.SemaphoreType.DMA,
                       pltpu.SemaphoreType.DMA,
                       pltpu.SemaphoreType.DMA,
                       pltpu.SemaphoreType.DMA,
                       pltpu.SemaphoreType.DMA,
                       pltpu.SemaphoreType.DMA,
                       pltpu.SemaphoreType.DMA,
                       pltpu.SemaphoreType.DMA,
                       pltpu.SemaphoreType.DMA,
                       pltpu.SemaphoreType.DMA],
    )
    S0, S1, cntm = sc_fn(A0, A1, B0, B1, C0, C1, dst, src, zzh, zc8, ones2)
    S0, S1, cntm = S0[:N], S1[:N], cntm[:N]

    G4 = 5
    NB4 = N // G4
    out = pl.pallas_call(
        functools.partial(_post_body, nsteps=G4, nb=NB4),
        grid=(G4,),
        in_specs=[rows(NB4, H), rows(NB4, H), rows(NB4, 8), rows(NB4, 256),
                  rows(NB4, 16), rows(NB4, 1), full(H, 256), full(H, 256),
                  full(1, 256), full(256, 256), full(16, 256), full(1, 256),
                  full(1, 256), full(1, 256)],
        out_specs=pl.BlockSpec((B, 256), lambda i: (0, 0)),
        out_shape=jax.ShapeDtypeStruct((B, 256), f32),
        scratch_shapes=[pltpu.VMEM((B, 256), f32),
                        pltpu.VMEM((B, 16), f32),
                        pltpu.VMEM((B, 8), f32)],
    )(S0, S1, cntm, s_full, vn, bt2, Wm2[:H], Wm2[H:], bm2[None, :],
      Wo[:256], Wo[256:], bo[None, :], gamma[None, :], beta[None, :])
    return out
